# Initial kernel scaffold; baseline (speedup 1.0000x reference)
import jax
import jax.numpy as jnp
from jax.experimental import pallas as pl


def kernel(hidden_states, gate_w, w1, w3, w2):
    raise NotImplementedError("write your pallas kernel here")


# speedup vs baseline: 1.5431x; 1.5431x over previous
import functools

import jax
import jax.numpy as jnp
from jax.experimental import pallas as pl
from jax.experimental.pallas import tpu as pltpu

T = 2048
D = 1024
FF = 4096
E = 8
K = 2
BK = 256
NB = 24
P = T * K
CH = 512
NF = 2

_f32 = jnp.float32
_bf16 = jnp.bfloat16
_i32 = jnp.int32


def _dot(a, b):
    return jax.lax.dot_general(a, b, (((1,), (0,)), ((), ())),
                               preferred_element_type=_f32)


def _router_body(x_ref, gw_ref, logits_ref, pos_ref, wgt_ref, meta_ref):
    x = x_ref[...]
    logits = jax.lax.dot_general(
        x, gw_ref[...], (((1,), (0,)), ((), ())),
        preferred_element_type=_f32, precision=jax.lax.Precision.DEFAULT)
    logits_ref[...] = logits

    m = jnp.max(logits, axis=1, keepdims=True)
    p = jnp.exp(logits - m)
    probs = p / jnp.sum(p, axis=1, keepdims=True)

    bw = probs[:, 0:1]
    bi = jnp.zeros((T, 1), _i32)
    for e in range(1, E):
        c = probs[:, e:e + 1]
        upd = c > bw
        bi = jnp.where(upd, e, bi)
        bw = jnp.where(upd, c, bw)
    sw = jnp.full((T, 1), -1.0, _f32)
    si = jnp.zeros((T, 1), _i32)
    for e in range(E):
        c = probs[:, e:e + 1]
        upd = jnp.logical_and(bi != e, c > sw)
        si = jnp.where(upd, e, si)
        sw = jnp.where(upd, c, sw)
    tot = bw + sw
    w0 = bw / tot
    w1v = sw / tot

    er = jax.lax.broadcasted_iota(_i32, (1, E), 1)
    oh0 = (bi == er).astype(_f32)
    oh1 = (si == er).astype(_f32)
    counts = jnp.sum(oh0 + oh1, axis=0, keepdims=True)
    blocks = jnp.ceil(counts * (1.0 / BK))
    u_strict = (jax.lax.broadcasted_iota(_i32, (E, E), 0)
                < jax.lax.broadcasted_iota(_i32, (E, E), 1)).astype(_f32)
    start = _dot(blocks, u_strict)
    slot_off = start * BK
    nact = start[:, E - 1:E] + blocks[:, E - 1:E]

    ohcat = jnp.concatenate([oh0, oh1], axis=0)
    offcat = jnp.sum(ohcat * slot_off, axis=1, keepdims=True)
    ltri = (jax.lax.broadcasted_iota(_i32, (CH, CH), 0)
            > jax.lax.broadcasted_iota(_i32, (CH, CH), 1)).astype(_bf16)
    carry = jnp.zeros((1, E), _f32)
    ranks = []
    for c in range(P // CH):
        oc = ohcat[c * CH:(c + 1) * CH]
        within = _dot(ltri, oc.astype(_bf16)) + carry
        ranks.append(jnp.sum(oc * within, axis=1, keepdims=True))
        carry = carry + jnp.sum(oc, axis=0, keepdims=True)
    rankv = jnp.concatenate(ranks, axis=0)
    pos_ref[...] = (offcat + rankv).astype(_i32)
    wgt_ref[...] = jnp.concatenate([w0, w1v], axis=0)

    jm = jax.lax.broadcasted_iota(_i32, (1, 32), 1)
    nact_i = nact.astype(_i32)
    jc = jnp.minimum(jnp.maximum(jm - 1, 0), nact_i - 1)
    starti = start.astype(_i32)
    be = -jnp.ones((1, 32), _i32)
    for e in range(E):
        be = be + (starti[:, e:e + 1] <= jc).astype(_i32)
    meta_ref[...] = jnp.where(jm == 0, nact_i, be)


def _mlp_body(meta_ref, xbf_ref, posr_ref, wgtr_ref,
              w1_ref, w3_ref, w2_ref, out_ref, xs_ref, yacc_ref):
    j = pl.program_id(0)
    f = pl.program_id(1)

    @pl.when(jnp.logical_and(j == 0, f == 0))
    def _init():
        out_ref[...] = jnp.zeros_like(out_ref)

    @pl.when(j < meta_ref[0])
    def _compute():
        base = j * BK
        posr = posr_ref[...]
        ii = jax.lax.broadcasted_iota(_i32, (BK, T), 0) + base

        @pl.when(f == 0)
        def _gather():
            g = ((posr[0:1, :] == ii).astype(_f32)
                 + (posr[1:2, :] == ii).astype(_f32)).astype(_bf16)
            xs_ref[...] = _dot(g, xbf_ref[...]).astype(_bf16)

        xs = xs_ref[...]
        a = _dot(xs, w1_ref[0])
        bv = _dot(xs, w3_ref[0])
        h = (a * jax.nn.sigmoid(a) * bv).astype(_bf16)
        yp = _dot(h, w2_ref[0])

        @pl.when(f == 0)
        def _y0():
            yacc_ref[...] = yp

        @pl.when(f > 0)
        def _y1():
            yacc_ref[...] += yp

        @pl.when(f == NF - 1)
        def _scatter():
            wgtr = wgtr_ref[...]
            st = (jnp.where(posr[0:1, :] == ii, wgtr[0:1, :], 0.0)
                  + jnp.where(posr[1:2, :] == ii, wgtr[1:2, :], 0.0)
                  ).astype(_bf16)
            out_ref[...] += jax.lax.dot_general(
                st, yacc_ref[...].astype(_bf16), (((0,), (0,)), ((), ())),
                preferred_element_type=_f32)


def _moe(hidden_states, gate_w, w1, w3, w2, interpret=False):
    b, s, d = hidden_states.shape
    x = hidden_states.reshape(T, D)

    logits, posp, wgtp, bmeta = pl.pallas_call(
        _router_body,
        out_shape=(
            jax.ShapeDtypeStruct((T, E), _f32),
            jax.ShapeDtypeStruct((P, 1), _i32),
            jax.ShapeDtypeStruct((P, 1), _f32),
            jax.ShapeDtypeStruct((1, 32), _i32),
        ),
        interpret=interpret,
    )(x, gate_w)

    meta = bmeta.reshape(32)
    pos_r = posp.reshape(K, T)
    wgt_r = wgtp.reshape(K, T)
    x_bf = x.astype(_bf16)
    w1_bf = w1.astype(_bf16)
    w3_bf = w3.astype(_bf16)
    w2_bf = w2.astype(_bf16)

    fh = FF // NF
    grid_spec = pltpu.PrefetchScalarGridSpec(
        num_scalar_prefetch=1,
        grid=(NB, NF),
        in_specs=[
            pl.BlockSpec((T, D), lambda j, f, m: (0, 0)),
            pl.BlockSpec((K, T), lambda j, f, m: (0, 0)),
            pl.BlockSpec((K, T), lambda j, f, m: (0, 0)),
            pl.BlockSpec((1, D, fh), lambda j, f, m: (m[1 + j], 0, f)),
            pl.BlockSpec((1, D, fh), lambda j, f, m: (m[1 + j], 0, f)),
            pl.BlockSpec((1, fh, D), lambda j, f, m: (m[1 + j], f, 0)),
        ],
        out_specs=pl.BlockSpec((T, D), lambda j, f, m: (0, 0)),
        scratch_shapes=[
            pltpu.VMEM((BK, D), _bf16),
            pltpu.VMEM((BK, D), _f32),
        ],
    )
    final = pl.pallas_call(
        _mlp_body,
        grid_spec=grid_spec,
        out_shape=jax.ShapeDtypeStruct((T, D), _f32),
        interpret=interpret,
    )(meta, x_bf, pos_r, wgt_r, w1_bf, w3_bf, w2_bf)

    return final.reshape(b, s, d), logits


def kernel(hidden_states, gate_w, w1, w3, w2):
    return _moe(hidden_states, gate_w, w1, w3, w2)
